# baseline (speedup 1.0000x reference)
import functools

import jax
import jax.numpy as jnp
from jax import lax
from jax.experimental import pallas as pl
from jax.experimental.pallas import tpu as pltpu
from jax.experimental.pallas import tpu_sc as plsc

T_ROWS = 1000
D_IN = 128
D_OUT = 512
B = 16384

_info = plsc.get_sparse_core_info()
NC, NS = _info.num_cores, _info.num_subcores
NW = NC * NS
BPW = B // NW
C = 64
NCHUNK = BPW // C
NBUF = 3


def _table_body(pe_ref, w1_ref, b1_ref, w2_ref, b2_ref, out_ref):
    h = jnp.dot(pe_ref[...], w1_ref[...], preferred_element_type=jnp.float32)
    h = h + b1_ref[...]
    h = h * jax.nn.sigmoid(h)
    out_ref[...] = (
        jnp.dot(h, w2_ref[...], preferred_element_type=jnp.float32) + b2_ref[...]
    )


def _compute_table(pe, W1, b1, W2, b2):
    return pl.pallas_call(
        _table_body,
        out_shape=jax.ShapeDtypeStruct((T_ROWS, D_OUT), jnp.float32),
    )(pe, W1, b1.reshape(1, D_OUT), W2, b2.reshape(1, D_OUT))


_mesh = plsc.VectorSubcoreMesh(core_axis_name="c", subcore_axis_name="s")


@functools.partial(
    pl.kernel,
    mesh=_mesh,
    compiler_params=pltpu.CompilerParams(
        disable_bounds_checks=True, disable_semaphore_checks=True
    ),
    out_type=jax.ShapeDtypeStruct((B, D_OUT), jnp.float32),
    scratch_types=[
        pltpu.VMEM((BPW,), jnp.int32),
        *[pltpu.VMEM((C, D_OUT), jnp.float32) for _ in range(NBUF)],
        *[pltpu.SemaphoreType.DMA for _ in range(2 * NBUF)],
    ],
)
def _sc_gather(table_hbm, idx_hbm, out_hbm, idx_v, *rest):
    bufs = rest[:NBUF]
    gsems = rest[NBUF:2 * NBUF]
    ssems = rest[2 * NBUF:]
    wid = lax.axis_index("s") * NC + lax.axis_index("c")
    base = wid * BPW
    pltpu.sync_copy(idx_hbm.at[pl.ds(base, C)], idx_v.at[pl.ds(0, C)])
    la = NBUF - 1
    g = [None] * NCHUNK
    s = [None] * NCHUNK
    g[0] = pltpu.async_copy(
        table_hbm.at[idx_v.at[pl.ds(0, C)]], bufs[0], gsems[0]
    )
    pltpu.sync_copy(
        idx_hbm.at[pl.ds(base + C, BPW - C)], idx_v.at[pl.ds(C, BPW - C)]
    )
    for c in range(1, min(la, NCHUNK)):
        g[c] = pltpu.async_copy(
            table_hbm.at[idx_v.at[pl.ds(c * C, C)]], bufs[c % NBUF], gsems[c % NBUF]
        )
    for c in range(NCHUNK):
        g[c].wait()
        n = c + la
        if n < NCHUNK:
            if n - NBUF >= 0:
                s[n - NBUF].wait()
            g[n] = pltpu.async_copy(
                table_hbm.at[idx_v.at[pl.ds(n * C, C)]],
                bufs[n % NBUF],
                gsems[n % NBUF],
            )
        s[c] = pltpu.async_copy(
            bufs[c % NBUF], out_hbm.at[pl.ds(base + c * C, C)], ssems[c % NBUF]
        )
    for c in range(max(0, NCHUNK - NBUF), NCHUNK):
        s[c].wait()


def kernel(x, pe, W1, b1, W2, b2):
    table = _compute_table(pe, W1, b1, W2, b2)
    return _sc_gather(table, x.astype(jnp.int32))


# speedup vs baseline: 1.0025x; 1.0025x over previous
import functools

import jax
import jax.numpy as jnp
from jax import lax
from jax.experimental import pallas as pl
from jax.experimental.pallas import tpu as pltpu
from jax.experimental.pallas import tpu_sc as plsc

T_ROWS = 1000
D_IN = 128
D_OUT = 512
B = 16384

_info = plsc.get_sparse_core_info()
NC, NS = _info.num_cores, _info.num_subcores
NW = NC * NS
BPW = B // NW
C = 64
NCHUNK = BPW // C
NBUF = 3


def _table_body(pe_ref, w1_ref, b1_ref, w2_ref, b2_ref, out_ref):
    h = jnp.dot(pe_ref[...], w1_ref[...], preferred_element_type=jnp.float32)
    h = h + b1_ref[...]
    h = h * jax.nn.sigmoid(h)
    out_ref[...] = (
        jnp.dot(h, w2_ref[...], preferred_element_type=jnp.float32) + b2_ref[...]
    )


def _compute_table(pe, W1, b1, W2, b2):
    return pl.pallas_call(
        _table_body,
        out_shape=jax.ShapeDtypeStruct((T_ROWS, D_OUT), jnp.float32),
    )(pe, W1, b1.reshape(1, D_OUT), W2, b2.reshape(1, D_OUT))


_mesh = plsc.VectorSubcoreMesh(core_axis_name="c", subcore_axis_name="s")


@functools.partial(
    pl.kernel,
    mesh=_mesh,
    out_type=jax.ShapeDtypeStruct((B, D_OUT), jnp.float32),
    scratch_types=[
        pltpu.VMEM((BPW,), jnp.int32),
        *[pltpu.VMEM((C, D_OUT), jnp.float32) for _ in range(NBUF)],
        *[pltpu.SemaphoreType.DMA for _ in range(2 * NBUF)],
    ],
)
def _sc_gather(table_hbm, idx_hbm, out_hbm, idx_v, *rest):
    bufs = rest[:NBUF]
    gsems = rest[NBUF:2 * NBUF]
    ssems = rest[2 * NBUF:]
    wid = lax.axis_index("s") * NC + lax.axis_index("c")
    base = wid * BPW
    pltpu.sync_copy(idx_hbm.at[pl.ds(base, C)], idx_v.at[pl.ds(0, C)])
    la = NBUF - 1
    g = [None] * NCHUNK
    s = [None] * NCHUNK
    g[0] = pltpu.async_copy(
        table_hbm.at[idx_v.at[pl.ds(0, C)]], bufs[0], gsems[0]
    )
    pltpu.sync_copy(
        idx_hbm.at[pl.ds(base + C, BPW - C)], idx_v.at[pl.ds(C, BPW - C)]
    )
    for c in range(1, min(la, NCHUNK)):
        g[c] = pltpu.async_copy(
            table_hbm.at[idx_v.at[pl.ds(c * C, C)]], bufs[c % NBUF], gsems[c % NBUF]
        )
    for c in range(NCHUNK):
        g[c].wait()
        n = c + la
        if n < NCHUNK:
            if n - NBUF >= 0:
                s[n - NBUF].wait()
            g[n] = pltpu.async_copy(
                table_hbm.at[idx_v.at[pl.ds(n * C, C)]],
                bufs[n % NBUF],
                gsems[n % NBUF],
            )
        s[c] = pltpu.async_copy(
            bufs[c % NBUF], out_hbm.at[pl.ds(base + c * C, C)], ssems[c % NBUF]
        )
    for c in range(max(0, NCHUNK - NBUF), NCHUNK):
        s[c].wait()


def kernel(x, pe, W1, b1, W2, b2):
    table = _compute_table(pe, W1, b1, W2, b2)
    return _sc_gather(table, x.astype(jnp.int32))
